# Initial kernel scaffold; baseline (speedup 1.0000x reference)
import jax
import jax.numpy as jnp
from jax.experimental import pallas as pl


def kernel(support_features, support_labels, query_features):
    raise NotImplementedError("write your pallas kernel here")


# speedup vs baseline: 2.6395x; 2.6395x over previous
import jax
import jax.numpy as jnp
from jax.experimental import pallas as pl

T, NS, NW, NQ, D = 32, 320, 16, 240, 512


def _body(lab_ref, sup_ref, qry_ref, out_ref):
    lab = lab_ref[0]
    sup = sup_ref[0]
    q = qry_ref[0]

    wids = jax.lax.broadcasted_iota(jnp.int32, (NW, NS), 0)
    oh = jnp.where(wids == lab, 1.0, 0.0).astype(jnp.float32)
    cnt = jnp.sum(oh, axis=1, keepdims=True)
    psum = jax.lax.dot_general(
        oh, sup, (((1,), (0,)), ((), ())),
        preferred_element_type=jnp.float32,
        precision=jax.lax.Precision.HIGHEST)
    protos = psum / cnt

    wn = jnp.sum(protos * protos, axis=1, keepdims=True)
    scores = jax.lax.dot_general(
        protos, q, (((1,), (1,)), ((), ())),
        preferred_element_type=jnp.float32,
        precision=jax.lax.Precision.HIGHEST)
    d2 = wn - 2.0 * scores

    idx = jax.lax.broadcasted_iota(jnp.int32, (NW, NQ), 0)
    m = jnp.min(d2, axis=0, keepdims=True)
    pred = jnp.min(jnp.where(d2 == m, idx, NW), axis=0, keepdims=True)
    out_ref[0] = pred.astype(jnp.int32)


@jax.jit
def kernel(support_features, support_labels, query_features):
    labels3 = support_labels.reshape(T, 1, NS)
    out = pl.pallas_call(
        _body,
        grid=(T,),
        in_specs=[
            pl.BlockSpec((1, 1, NS), lambda t: (t, 0, 0)),
            pl.BlockSpec((1, NS, D), lambda t: (t, 0, 0)),
            pl.BlockSpec((1, NQ, D), lambda t: (t, 0, 0)),
        ],
        out_specs=pl.BlockSpec((1, 1, NQ), lambda t: (t, 0, 0)),
        out_shape=jax.ShapeDtypeStruct((T, 1, NQ), jnp.int32),
    )(labels3, support_features, query_features)
    return out.reshape(T, NQ)
